# baseline (speedup 1.0000x reference)
import functools

import jax
import jax.numpy as jnp
from jax import lax
from jax.experimental import pallas as pl
from jax.experimental.pallas import tpu as pltpu

_LN_EPS = 1e-5


def _fused_kernel(p_ref, w_ref, b_ref, o_ref, *, inv_e, tn):
    params = b_ref[...]
    for t in range(tn):
        acc = jnp.dot(p_ref[t].astype(jnp.bfloat16), w_ref[...],
                      preferred_element_type=jnp.float32)
        acc = acc + params[0:1]
        mean = jnp.sum(acc, axis=-1, keepdims=True) * inv_e
        sumsq = jnp.sum(acc * acc, axis=-1, keepdims=True) * inv_e
        var = jnp.maximum(sumsq - mean * mean, 0.0)
        normed = (acc - mean) * lax.rsqrt(var + _LN_EPS)
        out = normed * params[1:2] + params[2:3]
        o_ref[t] = jnp.transpose(out, (1, 0)).astype(o_ref.dtype)


def kernel(x, conv_w, conv_b, ln_g, ln_b):
    N, C, H, W = x.shape
    E = conv_w.shape[0]
    P = 16
    Hp, Wp = H // P, W // P
    HW = Hp * Wp
    K = C * P * P
    compute_dtype = jnp.bfloat16
    out_dtype = x.dtype

    patches = x.reshape(N, C, Hp, P, Wp, P)
    patches = jnp.transpose(patches, (0, 2, 4, 1, 3, 5)).reshape(N, HW, K)

    w2d = conv_w.reshape(E, K).T.astype(compute_dtype)
    params = jnp.stack([conv_b, ln_g, ln_b]).astype(jnp.float32)
    params = jnp.pad(params, ((0, 8 - 3), (0, 0)))

    tn = 4
    grid = (N // tn,)
    cost = pl.CostEstimate(
        flops=2 * N * HW * K * E,
        transcendentals=N * HW,
        bytes_accessed=(N * HW * K * 4 + K * E * 2 + 8 * E * 4
                        + N * E * HW * 4))

    out = pl.pallas_call(
        functools.partial(_fused_kernel, inv_e=1.0 / E, tn=tn),
        out_shape=jax.ShapeDtypeStruct((N, E, HW), out_dtype),
        grid=grid,
        in_specs=[
            pl.BlockSpec((tn, HW, K), lambda i: (i, 0, 0)),
            pl.BlockSpec((K, E), lambda i: (0, 0)),
            pl.BlockSpec((8, E), lambda i: (0, 0)),
        ],
        out_specs=pl.BlockSpec((tn, E, HW), lambda i: (i, 0, 0)),
        compiler_params=pltpu.CompilerParams(
            dimension_semantics=("parallel",),
            allow_input_fusion=[True, False, False],
            vmem_limit_bytes=96 * 1024 * 1024),
        cost_estimate=cost,
    )(patches, w2d, params)

    return out.reshape(N, E, Hp, Wp)


# speedup vs baseline: 1.0923x; 1.0923x over previous
import functools

import jax
import jax.numpy as jnp
from jax import lax
from jax.experimental import pallas as pl
from jax.experimental.pallas import tpu as pltpu

_LN_EPS = 1e-5


def _fused_kernel(p_ref, w_ref, b_ref, o_ref, *, inv_e, tn):
    params = b_ref[...]
    for t in range(tn):
        acc = jnp.dot(p_ref[t].astype(jnp.bfloat16), w_ref[...],
                      preferred_element_type=jnp.float32)
        acc = acc + params[0:1]
        mean = jnp.sum(acc, axis=-1, keepdims=True) * inv_e
        sumsq = jnp.sum(acc * acc, axis=-1, keepdims=True) * inv_e
        var = jnp.maximum(sumsq - mean * mean, 0.0)
        normed = (acc - mean) * lax.rsqrt(var + _LN_EPS)
        out = normed * params[1:2] + params[2:3]
        o_ref[t] = jnp.transpose(out, (1, 0)).astype(o_ref.dtype)


def kernel(x, conv_w, conv_b, ln_g, ln_b):
    N, C, H, W = x.shape
    E = conv_w.shape[0]
    P = 16
    Hp, Wp = H // P, W // P
    HW = Hp * Wp
    K = C * P * P
    compute_dtype = jnp.bfloat16
    out_dtype = x.dtype

    patches = x.astype(compute_dtype).reshape(N, C, Hp, P, Wp, P)
    patches = jnp.transpose(patches, (0, 2, 4, 1, 3, 5)).reshape(N, HW, K)

    w2d = conv_w.reshape(E, K).T.astype(compute_dtype)
    params = jnp.stack([conv_b, ln_g, ln_b]).astype(jnp.float32)
    params = jnp.pad(params, ((0, 8 - 3), (0, 0)))

    tn = 4
    grid = (N // tn,)
    cost = pl.CostEstimate(
        flops=2 * N * HW * K * E,
        transcendentals=N * HW,
        bytes_accessed=(N * HW * K * 4 + K * E * 2 + 8 * E * 4
                        + N * E * HW * 4))

    out = pl.pallas_call(
        functools.partial(_fused_kernel, inv_e=1.0 / E, tn=tn),
        out_shape=jax.ShapeDtypeStruct((N, E, HW), out_dtype),
        grid=grid,
        in_specs=[
            pl.BlockSpec((tn, HW, K), lambda i: (i, 0, 0)),
            pl.BlockSpec((K, E), lambda i: (0, 0)),
            pl.BlockSpec((8, E), lambda i: (0, 0)),
        ],
        out_specs=pl.BlockSpec((tn, E, HW), lambda i: (i, 0, 0)),
        compiler_params=pltpu.CompilerParams(
            dimension_semantics=("parallel",),
            vmem_limit_bytes=96 * 1024 * 1024),
        cost_estimate=cost,
    )(patches, w2d, params)

    return out.reshape(N, E, Hp, Wp)
